# baseline (speedup 1.0000x reference)
import functools
import math

import jax
import jax.numpy as jnp
from jax import lax
from jax.experimental import pallas as pl
from jax.experimental.pallas import tpu as pltpu
from jax.experimental.pallas import tpu_sc as plsc

B = 16384
D = 64
L = 16
NC, NS = 2, 16
NW = NC * NS
BPW = B // NW


def _sc_body(ids_hbm, mu_s_hbm, lv_s_hbm, mu_a_hbm, lv_a_hbm,
             eps_s_hbm, eps_a_hbm,
             lat_s_out, lat_a_out, mu_s_out, lv_s_out, mu_a_out, lv_a_out,
             idx_v, mu_v, lv_v, eps_v, sem_mu, sem_lv, sem_eps):
    wid = lax.axis_index("s") * NC + lax.axis_index("c")
    base = wid * BPW
    pltpu.sync_copy(ids_hbm.at[pl.ds(base, BPW)], idx_v)

    pairs = (
        (mu_s_hbm, lv_s_hbm, eps_s_hbm, lat_s_out, mu_s_out, lv_s_out),
        (mu_a_hbm, lv_a_hbm, eps_a_hbm, lat_a_out, mu_a_out, lv_a_out),
    )
    for mu_hbm, lv_hbm, eps_hbm, lat_out, mu_out, lv_out in pairs:
        cp_mu = pltpu.async_copy(mu_hbm.at[idx_v], mu_v, sem_mu)
        cp_lv = pltpu.async_copy(lv_hbm.at[idx_v], lv_v, sem_lv)
        cp_eps = pltpu.async_copy(eps_hbm.at[pl.ds(base, BPW)], eps_v, sem_eps)
        cp_mu.wait()
        cp_lv.wait()
        cp_eps.wait()

        def row(i, _):
            for j in range(D // L):
                sl = pl.ds(j * L, L)
                mu = mu_v[i, sl]
                lv = lv_v[i, sl]
                ep = eps_v[i, sl]
                eps_v[i, sl] = mu + ep * jnp.exp(lv * 0.5)
            return 0

        lax.fori_loop(0, BPW, row, 0)

        pltpu.sync_copy(eps_v, lat_out.at[pl.ds(base, BPW)])
        pltpu.sync_copy(mu_v, mu_out.at[pl.ds(base, BPW)])
        pltpu.sync_copy(lv_v, lv_out.at[pl.ds(base, BPW)])


@jax.jit
def kernel(instance_ids, weight_mu_shape, weight_logvar_shape,
           weight_mu_app, weight_logvar_app):
    ek = jax.random.key(42)
    ek1, ek2 = jax.random.split(ek)
    eps_s = jax.random.normal(ek1, (B, D), dtype=jnp.float32)
    eps_a = jax.random.normal(ek2, (B, D), dtype=jnp.float32)

    f32 = jnp.float32
    out_type = tuple(jax.ShapeDtypeStruct((B, D), f32) for _ in range(6))
    mesh = plsc.VectorSubcoreMesh(core_axis_name="c", subcore_axis_name="s")
    run = pl.kernel(
        _sc_body,
        out_type=out_type,
        mesh=mesh,
        compiler_params=pltpu.CompilerParams(use_tc_tiling_on_sc=False),
        scratch_types=[
            pltpu.VMEM((BPW,), jnp.int32),
            pltpu.VMEM((BPW, D), f32),
            pltpu.VMEM((BPW, D), f32),
            pltpu.VMEM((BPW, D), f32),
            pltpu.SemaphoreType.DMA,
            pltpu.SemaphoreType.DMA,
            pltpu.SemaphoreType.DMA,
        ],
    )
    lat_s, lat_a, mu_s, lv_s, mu_a, lv_a = run(
        instance_ids.astype(jnp.int32),
        weight_mu_shape, weight_logvar_shape,
        weight_mu_app, weight_logvar_app,
        eps_s, eps_a,
    )
    return (lat_s, lat_a, mu_s, lv_s, mu_a, lv_a)


# speedup vs baseline: 1.3273x; 1.3273x over previous
import functools
import math

import jax
import jax.numpy as jnp
from jax import lax
from jax.experimental import pallas as pl
from jax.experimental.pallas import tpu as pltpu
from jax.experimental.pallas import tpu_sc as plsc

B = 16384
D = 64
L = 16
NC, NS = 2, 16
NW = NC * NS
BPW = B // NW
CH = 256
K = 16


def _sc_body(ids_hbm, mu_s_hbm, lv_s_hbm, mu_a_hbm, lv_a_hbm,
             eps_s_hbm, eps_a_hbm,
             lat_s_out, lat_a_out, mu_s_out, lv_s_out, mu_a_out, lv_a_out,
             idx_v, mu_v, lv_v, eps_v, sem_mu, sem_lv, sem_eps):
    wid = lax.axis_index("s") * NC + lax.axis_index("c")
    base = wid * BPW
    pltpu.sync_copy(ids_hbm.at[pl.ds(base, BPW)], idx_v)

    def gather_rows(tab_hbm, buf, sem, ch0):
        def burst(c, _):
            r0 = c * K
            iv = idx_v[pl.ds(ch0 + r0, L)]
            cps = []
            for k in range(K):
                row = iv[k]
                cps.append(pltpu.async_copy(
                    tab_hbm.at[pl.ds(row, 1), :],
                    buf.at[pl.ds(r0 + k, 1), :], sem))
            for cp in cps:
                cp.wait()
            return 0
        lax.fori_loop(0, CH // K, burst, 0)

    pairs = (
        (mu_s_hbm, lv_s_hbm, eps_s_hbm, lat_s_out, mu_s_out, lv_s_out),
        (mu_a_hbm, lv_a_hbm, eps_a_hbm, lat_a_out, mu_a_out, lv_a_out),
    )
    for mu_hbm, lv_hbm, eps_hbm, lat_out, mu_out, lv_out in pairs:
        for ch in range(BPW // CH):
            ch0 = ch * CH
            row0 = base + ch0
            cp_eps = pltpu.async_copy(
                eps_hbm.at[pl.ds(row0, CH), :], eps_v, sem_eps)
            gather_rows(mu_hbm, mu_v, sem_mu, ch0)
            gather_rows(lv_hbm, lv_v, sem_lv, ch0)
            cp_eps.wait()

            def row(i, _):
                for j in range(D // L):
                    sl = pl.ds(j * L, L)
                    mu = mu_v[i, sl]
                    lv = lv_v[i, sl]
                    ep = eps_v[i, sl]
                    eps_v[i, sl] = mu + ep * jnp.exp(lv * 0.5)
                return 0

            lax.fori_loop(0, CH, row, 0)

            pltpu.sync_copy(eps_v, lat_out.at[pl.ds(row0, CH), :])
            pltpu.sync_copy(mu_v, mu_out.at[pl.ds(row0, CH), :])
            pltpu.sync_copy(lv_v, lv_out.at[pl.ds(row0, CH), :])


@jax.jit
def kernel(instance_ids, weight_mu_shape, weight_logvar_shape,
           weight_mu_app, weight_logvar_app):
    ek = jax.random.key(42)
    ek1, ek2 = jax.random.split(ek)
    eps_s = jax.random.normal(ek1, (B, D), dtype=jnp.float32)
    eps_a = jax.random.normal(ek2, (B, D), dtype=jnp.float32)

    f32 = jnp.float32
    out_type = tuple(jax.ShapeDtypeStruct((B, D), f32) for _ in range(6))
    mesh = plsc.VectorSubcoreMesh(core_axis_name="c", subcore_axis_name="s")
    run = pl.kernel(
        _sc_body,
        out_type=out_type,
        mesh=mesh,
        scratch_types=[
            pltpu.VMEM((BPW,), jnp.int32),
            pltpu.VMEM((CH, D), f32),
            pltpu.VMEM((CH, D), f32),
            pltpu.VMEM((CH, D), f32),
            pltpu.SemaphoreType.DMA,
            pltpu.SemaphoreType.DMA,
            pltpu.SemaphoreType.DMA,
        ],
    )
    lat_s, lat_a, mu_s, lv_s, mu_a, lv_a = run(
        instance_ids.astype(jnp.int32),
        weight_mu_shape, weight_logvar_shape,
        weight_mu_app, weight_logvar_app,
        eps_s, eps_a,
    )
    return (lat_s, lat_a, mu_s, lv_s, mu_a, lv_a)
